# baseline (speedup 1.0000x reference)
import functools

import jax
import jax.numpy as jnp
from jax import lax
from jax.experimental import pallas as pl
from jax.experimental.pallas import tpu as pltpu
from jax.experimental.pallas import tpu_sc as plsc

DIM = 128
B = 4
S = 4096
EPS = 1e-5
LANES = 16
VPR = DIM // LANES

NC = 2
NS = 16
NW = NC * NS
TOK = B * S
TPW = TOK // NW
CHUNK = 128
NCHUNK = TPW // CHUNK

_MAGIC = 0x5F3759DF


def _body(x_hbm, pos_hbm, tok_hbm, gam_hbm, bet_hbm, out_hbm,
          idx_a, idx_b, row_a, row_b, gam_v, bet_v, sem_a, sem_b):
    cid = lax.axis_index("c")
    sid = lax.axis_index("s")
    wid = sid * NC + cid
    base = pl.multiple_of(wid * TPW, TPW)

    pltpu.sync_copy(gam_hbm, gam_v)
    pltpu.sync_copy(bet_hbm, bet_v)

    idx = (idx_a, idx_b)
    row = (row_a, row_b)
    sem = (sem_a, sem_b)
    pbase = pl.multiple_of(base % S, TPW)

    gb = []
    for d in range(VPR):
        sl = pl.ds(d * LANES, LANES)
        gb.append(gam_v[sl])
        gb.append(bet_v[sl])
    lanes = lax.iota(jnp.int32, LANES)
    perms = tuple(lanes ^ k for k in (8, 4, 2, 1))
    carry0 = (tuple(gb), perms)

    def start(g):
        pltpu.sync_copy(x_hbm.at[pl.ds(base + g * CHUNK, CHUNK)], idx[g % 2])
        pltpu.sync_copy(pos_hbm.at[pl.ds(pbase + g * CHUNK, CHUNK)], row[g % 2])
        return pltpu.async_copy(tok_hbm.at[idx[g % 2]], row[g % 2], sem[g % 2],
                                add=True)

    copies = [start(0)]
    for g in range(NCHUNK):
        off = g * CHUNK
        if g + 1 < NCHUNK:
            copies.append(start(g + 1))
        copies[g].wait()
        row_v = row[g % 2]

        def body(i, carry, row_v=row_v):
            t = []
            for d in range(VPR):
                sl = pl.ds(d * LANES, LANES)
                t.append(row_v[i, sl])
            ssum = t[0] + t[1]
            for d in range(2, VPR):
                ssum = ssum + t[d]
            sq = t[0] * t[0] + t[1] * t[1]
            for d in range(2, VPR):
                sq = sq + t[d] * t[d]
            gb_c, perms_c = carry
            for perm in perms_c:
                ssum = ssum + ssum.at[perm].get(mode="promise_in_bounds")
                sq = sq + sq.at[perm].get(mode="promise_in_bounds")
            mean_v = ssum * (1.0 / DIM)
            vv = sq * (1.0 / DIM) - mean_v * mean_v + EPS
            bits = lax.bitcast_convert_type(vv, jnp.int32)
            seed = jnp.full((LANES,), _MAGIC, dtype=jnp.int32) - (bits >> 1)
            y = lax.bitcast_convert_type(seed, jnp.float32)
            half = vv * 0.5
            for _ in range(3):
                y = y * (1.5 - half * y * y)
            for d in range(VPR):
                sl = pl.ds(d * LANES, LANES)
                o = (t[d] - mean_v) * (y * gb_c[2 * d]) + gb_c[2 * d + 1]
                row_v[i, sl] = o
            return carry

        plsc.parallel_loop(0, CHUNK, 1, unroll=8, carry=carry0)(body)
        pltpu.sync_copy(row_v, out_hbm.at[pl.ds(base + off, CHUNK)])


def _run(xf, tok_table, pos_table, gamma, beta):
    mesh = plsc.VectorSubcoreMesh(core_axis_name="c", subcore_axis_name="s")
    fn = functools.partial(
        pl.kernel,
        out_type=jax.ShapeDtypeStruct((TOK, DIM), jnp.float32),
        mesh=mesh,
        scratch_types=[
            pltpu.VMEM((CHUNK,), jnp.int32),
            pltpu.VMEM((CHUNK,), jnp.int32),
            pltpu.VMEM((CHUNK, DIM), jnp.float32),
            pltpu.VMEM((CHUNK, DIM), jnp.float32),
            pltpu.VMEM((DIM,), jnp.float32),
            pltpu.VMEM((DIM,), jnp.float32),
            pltpu.SemaphoreType.DMA,
            pltpu.SemaphoreType.DMA,
        ],
    )(_body)
    return fn(xf, pos_table, tok_table, gamma, beta)


def kernel(x, tok_table, pos_table, gamma, beta):
    xf = x.reshape(TOK).astype(jnp.int32)
    out = _run(xf, tok_table, pos_table, gamma, beta)
    return out.reshape(B, S, DIM)


# speedup vs baseline: 1.4071x; 1.4071x over previous
import functools

import jax
import jax.numpy as jnp
from jax import lax
from jax.experimental import pallas as pl
from jax.experimental.pallas import tpu as pltpu
from jax.experimental.pallas import tpu_sc as plsc

DIM = 128
B = 4
S = 4096
EPS = 1e-5
LANES = 16
VPR = DIM // LANES

NC = 2
NS = 16
NW = NC * NS
TOK = B * S
TPW = TOK // NW
CHUNK = 128
NCHUNK = TPW // CHUNK

_MAGIC = 0x5F3759DF


def _body(x_hbm, pos_hbm, tok_hbm, gam_hbm, bet_hbm, out_hbm,
          idx_a, idx_b, row_a, row_b, gam_v, bet_v, sem_a, sem_b):
    cid = lax.axis_index("c")
    sid = lax.axis_index("s")
    wid = sid * NC + cid
    base = pl.multiple_of(wid * TPW, TPW)

    pltpu.sync_copy(gam_hbm, gam_v)
    pltpu.sync_copy(bet_hbm, bet_v)

    idx = (idx_a, idx_b)
    row = (row_a, row_b)
    sem = (sem_a, sem_b)
    pbase = pl.multiple_of(base % S, TPW)

    gb = []
    for d in range(VPR):
        sl = pl.ds(d * LANES, LANES)
        gb.append(gam_v[sl])
        gb.append(bet_v[sl])
    lanes = lax.iota(jnp.int32, LANES)
    perms = tuple(lanes ^ k for k in (8, 4, 2, 1))
    carry0 = (tuple(gb), perms)

    def start(g):
        pltpu.sync_copy(x_hbm.at[pl.ds(base + g * CHUNK, CHUNK)], idx[g % 2])
        pltpu.sync_copy(pos_hbm.at[pl.ds(pbase + g * CHUNK, CHUNK)], row[g % 2])
        return pltpu.async_copy(tok_hbm.at[idx[g % 2]], row[g % 2], sem[g % 2],
                                add=True)

    copies = [start(0)]
    for g in range(NCHUNK):
        off = g * CHUNK
        if g + 1 < NCHUNK:
            copies.append(start(g + 1))
        copies[g].wait()
        row_v = row[g % 2]

        def body(i, carry, row_v=row_v):
            gb_c, perms_c = carry
            t = row_v[i, pl.ds(0, LANES)]
            ssum = t
            sq = t * t
            for d in range(1, VPR):
                t = row_v[i, pl.ds(d * LANES, LANES)]
                ssum = ssum + t
                sq = sq + t * t
            for perm in perms_c:
                ssum = ssum + ssum.at[perm].get(mode="promise_in_bounds")
                sq = sq + sq.at[perm].get(mode="promise_in_bounds")
            mean_v = ssum * (1.0 / DIM)
            vv = sq * (1.0 / DIM) - mean_v * mean_v + EPS
            bits = lax.bitcast_convert_type(vv, jnp.int32)
            seed = jnp.full((LANES,), _MAGIC, dtype=jnp.int32) - (bits >> 1)
            y = lax.bitcast_convert_type(seed, jnp.float32)
            half = vv * 0.5
            for _ in range(2):
                y = y * (1.5 - half * y * y)
            for d in range(VPR):
                sl = pl.ds(d * LANES, LANES)
                o = (row_v[i, sl] - mean_v) * (y * gb_c[2 * d]) + gb_c[2 * d + 1]
                row_v[i, sl] = o
            return carry

        plsc.parallel_loop(0, CHUNK, 1, unroll=4, carry=carry0)(body)
        pltpu.sync_copy(row_v, out_hbm.at[pl.ds(base + off, CHUNK)])


def _run(xf, tok_table, pos_table, gamma, beta):
    mesh = plsc.VectorSubcoreMesh(core_axis_name="c", subcore_axis_name="s")
    fn = functools.partial(
        pl.kernel,
        out_type=jax.ShapeDtypeStruct((TOK, DIM), jnp.float32),
        mesh=mesh,
        scratch_types=[
            pltpu.VMEM((CHUNK,), jnp.int32),
            pltpu.VMEM((CHUNK,), jnp.int32),
            pltpu.VMEM((CHUNK, DIM), jnp.float32),
            pltpu.VMEM((CHUNK, DIM), jnp.float32),
            pltpu.VMEM((DIM,), jnp.float32),
            pltpu.VMEM((DIM,), jnp.float32),
            pltpu.SemaphoreType.DMA,
            pltpu.SemaphoreType.DMA,
        ],
    )(_body)
    return fn(xf, pos_table, tok_table, gamma, beta)


def kernel(x, tok_table, pos_table, gamma, beta):
    xf = x.reshape(TOK).astype(jnp.int32)
    out = _run(xf, tok_table, pos_table, gamma, beta)
    return out.reshape(B, S, DIM)
